# baseline (speedup 1.0000x reference)
import jax
import jax.numpy as jnp
from jax import lax
from jax.experimental import pallas as pl
from jax.experimental.pallas import tpu as pltpu

_B = 128
_N = 4096
_K = 8
_NBLK = 8
_BO = _N // _NBLK

_NEG = float("-inf")


def _body(field_ref, fblk_ref, w_ref, b_ref, out_ref, tops_ref, topf_ref):
    i = pl.program_id(0)
    f = field_ref[...]
    wblk = w_ref[...]
    blk = lax.dot_general(
        f, wblk, (((1,), (1,)), ((), ())), preferred_element_type=jnp.float32
    )
    s = blk + b_ref[...].reshape(1, _BO)
    fblk = fblk_ref[...]
    col = lax.broadcasted_iota(jnp.int32, (_B, _BO), 1)

    loc_s, loc_f = [], []
    for _ in range(_K):
        m = jnp.max(s, axis=1, keepdims=True)
        idx = jnp.min(jnp.where(s >= m, col, _BO), axis=1, keepdims=True)
        hit = col == idx
        loc_s.append(m)
        loc_f.append(jnp.max(jnp.where(hit, fblk, _NEG), axis=1, keepdims=True))
        s = jnp.where(hit, _NEG, s)

    run_s = jnp.where(i == 0, _NEG, tops_ref[...])
    run_f = topf_ref[...]
    cand_s = jnp.concatenate([run_s] + loc_s, axis=1)
    cand_f = jnp.concatenate([run_f] + loc_f, axis=1)

    ccol = lax.broadcasted_iota(jnp.int32, (_B, 2 * _K), 1)
    new_s, new_f = [], []
    for _ in range(_K):
        m = jnp.max(cand_s, axis=1, keepdims=True)
        idx = jnp.min(jnp.where(cand_s >= m, ccol, 2 * _K), axis=1, keepdims=True)
        hit = ccol == idx
        new_s.append(m)
        new_f.append(jnp.max(jnp.where(hit, cand_f, _NEG), axis=1, keepdims=True))
        cand_s = jnp.where(hit, _NEG, cand_s)

    tops_ref[...] = jnp.concatenate(new_s, axis=1)
    topf_ref[...] = jnp.concatenate(new_f, axis=1)

    @pl.when(i == _NBLK - 1)
    def _emit():
        out_ref[...] = topf_ref[...]


def kernel(field, W, b, training):
    del training
    return pl.pallas_call(
        _body,
        grid=(_NBLK,),
        in_specs=[
            pl.BlockSpec((_B, _N), lambda i: (0, 0)),
            pl.BlockSpec((_B, _BO), lambda i: (0, i)),
            pl.BlockSpec((_BO, _N), lambda i: (i, 0)),
            pl.BlockSpec((_BO,), lambda i: (i,)),
        ],
        out_specs=pl.BlockSpec((_B, _K), lambda i: (0, 0)),
        out_shape=jax.ShapeDtypeStruct((_B, _K), jnp.float32),
        scratch_shapes=[
            pltpu.VMEM((_B, _K), jnp.float32),
            pltpu.VMEM((_B, _K), jnp.float32),
        ],
        compiler_params=pltpu.CompilerParams(
            dimension_semantics=("arbitrary",),
        ),
    )(field, field, W, b)
